# baseline (speedup 1.0000x reference)
import functools

import jax
import jax.numpy as jnp
from jax import lax
from jax.experimental import pallas as pl
from jax.experimental.pallas import tpu as pltpu
from jax.experimental.pallas import tpu_sc as plsc

_B = 4096
_L = 200
_ND = 8
_F = 32
_VOCABS = (7, 24, 2, 100, 12, 31)
_NDEST = 1000000

_NC = 2
_NS = 16
_NW = _NC * _NS
_RPW = _B // _NW
_IPW = _RPW * _L
_NCH = _RPW
_GA = 128
_GB = _L - _GA
_NCC = (_L + 15) // 16
_TBLK = 4000


def _prep_body(users_ref, items_ref, dow_ref, time_ref, sex_ref, age_ref,
               month_ref, day_ref, w_ref, b_ref, u_ref, it_ref):
    tables = (dow_ref, time_ref, sex_ref, age_ref, month_ref, day_ref)
    u = jnp.broadcast_to(b_ref[...], (_B, _F))
    for k, (tbl, v) in enumerate(zip(tables, _VOCABS)):
        proj = jnp.dot(tbl[...], w_ref[k * _ND:(k + 1) * _ND, :],
                       preferred_element_type=jnp.float32)
        col = users_ref[:, k:k + 1]
        iota = lax.broadcasted_iota(jnp.int32, (_B, v), 1)
        onehot = (col == iota).astype(jnp.float32)
        u = u + jnp.dot(onehot, proj, preferred_element_type=jnp.float32)
    u_ref[:, : _F] = u
    u_ref[:, _F:] = jnp.zeros((_B, 128 - _F), jnp.float32)
    it = items_ref[...]
    q = it // (_NDEST // 4)
    packed = (it - q * (_NDEST // 4)) | lax.shift_left(q, 20)
    it_ref[0] = packed[:, : _GA]
    it_ref[1] = jnp.pad(packed[:, _GA:], ((0, 0), (0, 128 - _GB)))


def _prep(users, items, dow, time, sex, age, month, day, w, b):
    return pl.pallas_call(
        _prep_body,
        out_shape=(jax.ShapeDtypeStruct((_B, 128), jnp.float32),
                   jax.ShapeDtypeStruct((2, _B, 128), jnp.int32)),
    )(users, items, dow, time, sex, age, month, day, w, b.reshape(1, _F))


def _sc_repack(table):
    m = 1000
    nch = (_NDEST // 4) // m

    def body(x0, x1, x2, x3, o_ref):
        for kk, x in enumerate((x0, x1, x2, x3)):
            o_ref[:, _F * kk:_F * (kk + 1)] = x[...]

    return pl.pallas_call(
        body,
        grid=(nch,),
        in_specs=[pl.BlockSpec((m, _F), lambda i, kk=kk: (i + nch * kk, 0))
                  for kk in range(4)],
        out_specs=pl.BlockSpec((m, 128), lambda i: (i, 0)),
        out_shape=jax.ShapeDtypeStruct((_NDEST // 4, 128), jnp.float32),
    )(table, table, table, table)


def _sc_dot(u_pad, items3d, table4):
    mesh = plsc.VectorSubcoreMesh(core_axis_name="c", subcore_axis_name="s")

    @functools.partial(
        pl.kernel,
        out_type=jax.ShapeDtypeStruct((_B * _L,), jnp.float32),
        mesh=mesh,
        compiler_params=pltpu.CompilerParams(needs_layout_passes=False,
                                             use_tc_tiling_on_sc=False),
        scratch_types=[
            pltpu.VMEM((_RPW, _F), jnp.float32),
            pltpu.VMEM((_RPW, _GA), jnp.int32),
            pltpu.VMEM((_RPW + 1, _GB), jnp.int32),
            pltpu.VMEM((_L + 8,), jnp.int32),
            pltpu.VMEM((_L + 8,), jnp.int32),
            pltpu.VMEM((_L + 8,), jnp.int32),
            pltpu.VMEM((_L + 8,), jnp.int32),
            pltpu.VMEM((_L + 8, 128), jnp.float32),
            pltpu.VMEM((_L + 8, 128), jnp.float32),
            pltpu.VMEM((_L + 8,), jnp.float32),
            pltpu.VMEM((_L + 8,), jnp.float32),
            pltpu.SemaphoreType.DMA,
            pltpu.SemaphoreType.DMA,
            pltpu.SemaphoreType.DMA,
            pltpu.SemaphoreType.DMA,
        ],
    )
    def k(u_hbm, items_hbm, table_hbm, out_hbm,
          u_v, idx_a, idx_b, raw0, raw1, sh0, sh1, rows0, rows1, out0, out1,
          gs0, gs1, os0, os1):
        wid = lax.axis_index("s") * _NC + lax.axis_index("c")
        rbase = wid * _RPW
        lanes = lax.iota(jnp.int32, 16)

        pltpu.sync_copy(u_hbm.at[pl.ds(rbase, _RPW), pl.ds(0, _F)], u_v)
        pltpu.sync_copy(items_hbm.at[0, pl.ds(rbase, _RPW)], idx_a)
        pltpu.sync_copy(items_hbm.at[1, pl.ds(rbase, _RPW), pl.ds(0, _GB)],
                        idx_b.at[pl.ds(0, _RPW)])

        def transform_and_fire(c, raw, sh, rows, gs):
            rsplat = jnp.full((16,), c, jnp.int32)
            for cc in range(_NCC):
                if cc < 8:
                    itv = plsc.load_gather(idx_a, [rsplat, lanes + cc * 16])
                else:
                    itv = plsc.load_gather(idx_b,
                                           [rsplat, lanes + (cc - 8) * 16])
                raw[pl.ds(cc * 16, 16)] = itv
                sh[pl.ds(cc * 16, 16)] = itv & 0xFFFFF
            pltpu.async_copy(table_hbm.at[sh.at[pl.ds(0, _GA)]],
                             rows.at[pl.ds(0, _GA)], gs)
            pltpu.async_copy(table_hbm.at[sh.at[pl.ds(_GA, _GB)]],
                             rows.at[pl.ds(_GA, _GB)], gs)

        def wait_gathers(sh, rows, gs):
            pltpu.make_async_copy(table_hbm.at[sh.at[pl.ds(0, _GA)]],
                                  rows.at[pl.ds(0, _GA)], gs).wait()
            pltpu.make_async_copy(table_hbm.at[sh.at[pl.ds(_GA, _GB)]],
                                  rows.at[pl.ds(_GA, _GB)], gs).wait()

        def compute(c, raw, rows, out_v):
            rsplat = jnp.full((16,), c, jnp.int32)
            u_lo = plsc.load_gather(u_v, [rsplat, lanes])
            u_hi = plsc.load_gather(u_v, [rsplat, lanes + 16])

            @plsc.parallel_loop(0, _NCC, 1)
            def cch(cc):
                base = cc * 16
                ridx = base + lanes
                itv = raw[pl.ds(base, 16)]
                sub = lax.shift_right_logical(itv, 15) & 96
                accs = [jnp.zeros((16,), jnp.float32) for _ in range(4)]
                for f in range(_F):
                    src = u_lo if f < 16 else u_hi
                    ub = src.at[jnp.full((16,), f % 16, jnp.int32)].get(
                        mode="promise_in_bounds")
                    vals = plsc.load_gather(rows, [ridx, sub + f])
                    accs[f % 4] = accs[f % 4] + ub * vals
                out_v[pl.ds(base, 16)] = ((accs[0] + accs[1])
                                          + (accs[2] + accs[3]))

        def slot(c, raw, sh, rows, out_v, raw_n, sh_n, rows_n,
                 gs_mine, gs_next, os_mine):
            pl.when(c + 1 < _NCH)(
                lambda: transform_and_fire(c + 1, raw_n, sh_n, rows_n,
                                           gs_next))
            wait_gathers(sh, rows, gs_mine)
            pl.when(c >= 2)(lambda: pltpu.make_async_copy(
                out_v.at[pl.ds(0, _L)],
                out_hbm.at[pl.ds(wid * _IPW, _L)], os_mine).wait())
            compute(c, raw, rows, out_v)
            pltpu.async_copy(out_v.at[pl.ds(0, _L)],
                             out_hbm.at[pl.ds(wid * _IPW + c * _L, _L)],
                             os_mine)

        transform_and_fire(0, raw0, sh0, rows0, gs0)

        def pair(i, carry):
            c = 2 * i
            slot(c, raw0, sh0, rows0, out0, raw1, sh1, rows1, gs0, gs1, os0)
            slot(c + 1, raw1, sh1, rows1, out1, raw0, sh0, rows0,
                 gs1, gs0, os1)
            return carry

        lax.fori_loop(0, _NCH // 2, pair, 0)
        pltpu.make_async_copy(out0.at[pl.ds(0, _L)],
                              out_hbm.at[pl.ds(wid * _IPW, _L)], os0).wait()
        pltpu.make_async_copy(out1.at[pl.ds(0, _L)],
                              out_hbm.at[pl.ds(wid * _IPW, _L)], os1).wait()

    return k(u_pad, items3d, table4)


def kernel(users, items, dow_emb, time_emb, sex_emb, age_emb, month_emb,
           day_emb, W, b, item_table):
    users = users.astype(jnp.int32)
    items = items.astype(jnp.int32)
    u_pad, items3d = _prep(users, items, dow_emb, time_emb, sex_emb, age_emb,
                           month_emb, day_emb, W, b)
    table4 = _sc_repack(item_table)
    out = _sc_dot(u_pad, items3d, table4)
    return out.reshape(_B, _L)


# speedup vs baseline: 1.3803x; 1.3803x over previous
import functools

import jax
import jax.numpy as jnp
from jax import lax
from jax.experimental import pallas as pl
from jax.experimental.pallas import tpu as pltpu
from jax.experimental.pallas import tpu_sc as plsc

_B = 4096
_L = 200
_ND = 8
_F = 32
_VOCABS = (7, 24, 2, 100, 12, 31)
_NDEST = 1000000
_QS = 256000

_NC = 2
_NS = 16
_NW = _NC * _NS
_RPW = _B // _NW
_IPW = _RPW * _L
_NCH = _RPW
_GA = 128
_GB = _L - _GA
_NCC = (_L + 15) // 16
_TBLK = 4000


def _prep_body(users_ref, items_ref, dow_ref, time_ref, sex_ref, age_ref,
               month_ref, day_ref, w_ref, b_ref, u_ref, it_ref):
    tables = (dow_ref, time_ref, sex_ref, age_ref, month_ref, day_ref)
    u = jnp.broadcast_to(b_ref[...], (_B, _F))
    for k, (tbl, v) in enumerate(zip(tables, _VOCABS)):
        proj = jnp.dot(tbl[...], w_ref[k * _ND:(k + 1) * _ND, :],
                       preferred_element_type=jnp.float32)
        col = users_ref[:, k:k + 1]
        iota = lax.broadcasted_iota(jnp.int32, (_B, v), 1)
        onehot = (col == iota).astype(jnp.float32)
        u = u + jnp.dot(onehot, proj, preferred_element_type=jnp.float32)
    u_ref[:, : _F] = u
    u_ref[:, _F:] = jnp.zeros((_B, 128 - _F), jnp.float32)
    it = items_ref[...]
    q = it // _QS
    packed = (it - q * _QS) | lax.shift_left(q, 20)
    it_ref[0] = packed[:, : _GA]
    it_ref[1] = jnp.pad(packed[:, _GA:], ((0, 0), (0, 128 - _GB)))


def _prep(users, items, dow, time, sex, age, month, day, w, b):
    return pl.pallas_call(
        _prep_body,
        out_shape=(jax.ShapeDtypeStruct((_B, 128), jnp.float32),
                   jax.ShapeDtypeStruct((2, _B, 128), jnp.int32)),
    )(users, items, dow, time, sex, age, month, day, w, b.reshape(1, _F))


def _sc_repack(table):
    m = 1024
    nch = _QS // m
    nblk = _NDEST // m
    table_t = table.T

    def body(x0, x1, x2, x3, o_ref):
        for kk, x in enumerate((x0, x1, x2, x3)):
            o_ref[:, _F * kk:_F * (kk + 1)] = x[...].T

    return pl.pallas_call(
        body,
        grid=(nch,),
        in_specs=[pl.BlockSpec(
            (_F, m), lambda i, kk=kk: (0, jnp.minimum(i + nch * kk, nblk)))
            for kk in range(4)],
        out_specs=pl.BlockSpec((m, 128), lambda i: (i, 0)),
        out_shape=jax.ShapeDtypeStruct((_QS, 128), jnp.float32),
    )(table_t, table_t, table_t, table_t)


def _sc_dot(u_pad, items3d, table4):
    mesh = plsc.VectorSubcoreMesh(core_axis_name="c", subcore_axis_name="s")

    @functools.partial(
        pl.kernel,
        out_type=jax.ShapeDtypeStruct((_B * _L,), jnp.float32),
        mesh=mesh,
        compiler_params=pltpu.CompilerParams(needs_layout_passes=False,
                                             use_tc_tiling_on_sc=False),
        scratch_types=[
            pltpu.VMEM((_RPW, _F), jnp.float32),
            pltpu.VMEM((_RPW, _GA), jnp.int32),
            pltpu.VMEM((_RPW + 1, _GB), jnp.int32),
            pltpu.VMEM((_L + 8,), jnp.int32),
            pltpu.VMEM((_L + 8,), jnp.int32),
            pltpu.VMEM((_L + 8,), jnp.int32),
            pltpu.VMEM((_L + 8,), jnp.int32),
            pltpu.VMEM((_L + 8, 128), jnp.float32),
            pltpu.VMEM((_L + 8, 128), jnp.float32),
            pltpu.VMEM((_L + 8,), jnp.float32),
            pltpu.VMEM((_L + 8,), jnp.float32),
            pltpu.SemaphoreType.DMA,
            pltpu.SemaphoreType.DMA,
            pltpu.SemaphoreType.DMA,
            pltpu.SemaphoreType.DMA,
        ],
    )
    def k(u_hbm, items_hbm, table_hbm, out_hbm,
          u_v, idx_a, idx_b, raw0, raw1, sh0, sh1, rows0, rows1, out0, out1,
          gs0, gs1, os0, os1):
        wid = lax.axis_index("s") * _NC + lax.axis_index("c")
        rbase = wid * _RPW
        lanes = lax.iota(jnp.int32, 16)

        pltpu.sync_copy(u_hbm.at[pl.ds(rbase, _RPW), pl.ds(0, _F)], u_v)
        pltpu.sync_copy(items_hbm.at[0, pl.ds(rbase, _RPW)], idx_a)
        pltpu.sync_copy(items_hbm.at[1, pl.ds(rbase, _RPW), pl.ds(0, _GB)],
                        idx_b.at[pl.ds(0, _RPW)])

        def transform_and_fire(c, raw, sh, rows, gs):
            rsplat = jnp.full((16,), c, jnp.int32)
            for cc in range(_NCC):
                if cc < 8:
                    itv = plsc.load_gather(idx_a, [rsplat, lanes + cc * 16])
                else:
                    itv = plsc.load_gather(idx_b,
                                           [rsplat, lanes + (cc - 8) * 16])
                raw[pl.ds(cc * 16, 16)] = itv
                sh[pl.ds(cc * 16, 16)] = itv & 0xFFFFF
            pltpu.async_copy(table_hbm.at[sh.at[pl.ds(0, _GA)]],
                             rows.at[pl.ds(0, _GA)], gs)
            pltpu.async_copy(table_hbm.at[sh.at[pl.ds(_GA, _GB)]],
                             rows.at[pl.ds(_GA, _GB)], gs)

        def wait_gathers(sh, rows, gs):
            pltpu.make_async_copy(table_hbm.at[sh.at[pl.ds(0, _GA)]],
                                  rows.at[pl.ds(0, _GA)], gs).wait()
            pltpu.make_async_copy(table_hbm.at[sh.at[pl.ds(_GA, _GB)]],
                                  rows.at[pl.ds(_GA, _GB)], gs).wait()

        def compute(c, raw, rows, out_v):
            rsplat = jnp.full((16,), c, jnp.int32)
            u_lo = plsc.load_gather(u_v, [rsplat, lanes])
            u_hi = plsc.load_gather(u_v, [rsplat, lanes + 16])

            @plsc.parallel_loop(0, _NCC, 1)
            def cch(cc):
                base = cc * 16
                ridx = base + lanes
                itv = raw[pl.ds(base, 16)]
                sub = lax.shift_right_logical(itv, 15) & 96
                accs = [jnp.zeros((16,), jnp.float32) for _ in range(4)]
                for f in range(_F):
                    src = u_lo if f < 16 else u_hi
                    ub = src.at[jnp.full((16,), f % 16, jnp.int32)].get(
                        mode="promise_in_bounds")
                    vals = plsc.load_gather(rows, [ridx, sub + f])
                    accs[f % 4] = accs[f % 4] + ub * vals
                out_v[pl.ds(base, 16)] = ((accs[0] + accs[1])
                                          + (accs[2] + accs[3]))

        def slot(c, raw, sh, rows, out_v, raw_n, sh_n, rows_n,
                 gs_mine, gs_next, os_mine):
            pl.when(c + 1 < _NCH)(
                lambda: transform_and_fire(c + 1, raw_n, sh_n, rows_n,
                                           gs_next))
            wait_gathers(sh, rows, gs_mine)
            pl.when(c >= 2)(lambda: pltpu.make_async_copy(
                out_v.at[pl.ds(0, _L)],
                out_hbm.at[pl.ds(wid * _IPW, _L)], os_mine).wait())
            compute(c, raw, rows, out_v)
            pltpu.async_copy(out_v.at[pl.ds(0, _L)],
                             out_hbm.at[pl.ds(wid * _IPW + c * _L, _L)],
                             os_mine)

        transform_and_fire(0, raw0, sh0, rows0, gs0)

        def pair(i, carry):
            c = 2 * i
            slot(c, raw0, sh0, rows0, out0, raw1, sh1, rows1, gs0, gs1, os0)
            slot(c + 1, raw1, sh1, rows1, out1, raw0, sh0, rows0,
                 gs1, gs0, os1)
            return carry

        lax.fori_loop(0, _NCH // 2, pair, 0)
        pltpu.make_async_copy(out0.at[pl.ds(0, _L)],
                              out_hbm.at[pl.ds(wid * _IPW, _L)], os0).wait()
        pltpu.make_async_copy(out1.at[pl.ds(0, _L)],
                              out_hbm.at[pl.ds(wid * _IPW, _L)], os1).wait()

    return k(u_pad, items3d, table4)


def kernel(users, items, dow_emb, time_emb, sex_emb, age_emb, month_emb,
           day_emb, W, b, item_table):
    users = users.astype(jnp.int32)
    items = items.astype(jnp.int32)
    u_pad, items3d = _prep(users, items, dow_emb, time_emb, sex_emb, age_emb,
                           month_emb, day_emb, W, b)
    table4 = _sc_repack(item_table)
    out = _sc_dot(u_pad, items3d, table4)
    return out.reshape(_B, _L)


# speedup vs baseline: 1.3805x; 1.0001x over previous
import functools

import jax
import jax.numpy as jnp
from jax import lax
from jax.experimental import pallas as pl
from jax.experimental.pallas import tpu as pltpu
from jax.experimental.pallas import tpu_sc as plsc

_B = 4096
_L = 200
_ND = 8
_F = 32
_VOCABS = (7, 24, 2, 100, 12, 31)
_NDEST = 1000000
_QS = 256000

_NC = 2
_NS = 16
_NW = _NC * _NS
_RPW = _B // _NW
_IPW = _RPW * _L
_NCH = _RPW
_GA = 128
_GB = _L - _GA
_NCC = (_L + 15) // 16
_TBLK = 4000


def _prep_body(users_ref, items_ref, dow_ref, time_ref, sex_ref, age_ref,
               month_ref, day_ref, w_ref, b_ref, u_ref, it_ref):
    tables = (dow_ref, time_ref, sex_ref, age_ref, month_ref, day_ref)
    u = jnp.broadcast_to(b_ref[...], (_B, _F))
    for k, (tbl, v) in enumerate(zip(tables, _VOCABS)):
        proj = jnp.dot(tbl[...], w_ref[k * _ND:(k + 1) * _ND, :],
                       preferred_element_type=jnp.float32)
        col = users_ref[:, k:k + 1]
        iota = lax.broadcasted_iota(jnp.int32, (_B, v), 1)
        onehot = (col == iota).astype(jnp.float32)
        u = u + jnp.dot(onehot, proj, preferred_element_type=jnp.float32)
    u_ref[:, : _F] = u
    u_ref[:, _F:] = jnp.zeros((_B, 128 - _F), jnp.float32)
    it = items_ref[...]
    q = it // _QS
    packed = (it - q * _QS) | lax.shift_left(q, 20)
    it_ref[0] = packed[:, : _GA]
    it_ref[1] = jnp.pad(packed[:, _GA:], ((0, 0), (0, 128 - _GB)))


def _prep(users, items, dow, time, sex, age, month, day, w, b):
    return pl.pallas_call(
        _prep_body,
        out_shape=(jax.ShapeDtypeStruct((_B, 128), jnp.float32),
                   jax.ShapeDtypeStruct((2, _B, 128), jnp.int32)),
    )(users, items, dow, time, sex, age, month, day, w, b.reshape(1, _F))


def _sc_repack(table):
    m = 1024
    nch = _QS // m
    nblk = _NDEST // m
    table_t = table.T

    def body(x0, x1, x2, x3, o_ref):
        o_ref[...] = jnp.concatenate(
            [x[...].T for x in (x0, x1, x2, x3)], axis=1)

    return pl.pallas_call(
        body,
        grid=(nch,),
        in_specs=[pl.BlockSpec(
            (_F, m), lambda i, kk=kk: (0, jnp.minimum(i + nch * kk, nblk)))
            for kk in range(4)],
        out_specs=pl.BlockSpec((m, 128), lambda i: (i, 0)),
        out_shape=jax.ShapeDtypeStruct((_QS, 128), jnp.float32),
    )(table_t, table_t, table_t, table_t)


def _sc_dot(u_pad, items3d, table4):
    mesh = plsc.VectorSubcoreMesh(core_axis_name="c", subcore_axis_name="s")

    @functools.partial(
        pl.kernel,
        out_type=jax.ShapeDtypeStruct((_B * _L,), jnp.float32),
        mesh=mesh,
        compiler_params=pltpu.CompilerParams(needs_layout_passes=False,
                                             use_tc_tiling_on_sc=False),
        scratch_types=[
            pltpu.VMEM((_RPW, _F), jnp.float32),
            pltpu.VMEM((_RPW, _GA), jnp.int32),
            pltpu.VMEM((_RPW + 1, _GB), jnp.int32),
            pltpu.VMEM((_L + 8,), jnp.int32),
            pltpu.VMEM((_L + 8,), jnp.int32),
            pltpu.VMEM((_L + 8,), jnp.int32),
            pltpu.VMEM((_L + 8,), jnp.int32),
            pltpu.VMEM((_L + 8, 128), jnp.float32),
            pltpu.VMEM((_L + 8, 128), jnp.float32),
            pltpu.VMEM((_L + 8,), jnp.float32),
            pltpu.VMEM((_L + 8,), jnp.float32),
            pltpu.SemaphoreType.DMA,
            pltpu.SemaphoreType.DMA,
            pltpu.SemaphoreType.DMA,
            pltpu.SemaphoreType.DMA,
        ],
    )
    def k(u_hbm, items_hbm, table_hbm, out_hbm,
          u_v, idx_a, idx_b, raw0, raw1, sh0, sh1, rows0, rows1, out0, out1,
          gs0, gs1, os0, os1):
        wid = lax.axis_index("s") * _NC + lax.axis_index("c")
        rbase = wid * _RPW
        lanes = lax.iota(jnp.int32, 16)

        pltpu.sync_copy(u_hbm.at[pl.ds(rbase, _RPW), pl.ds(0, _F)], u_v)
        pltpu.sync_copy(items_hbm.at[0, pl.ds(rbase, _RPW)], idx_a)
        pltpu.sync_copy(items_hbm.at[1, pl.ds(rbase, _RPW), pl.ds(0, _GB)],
                        idx_b.at[pl.ds(0, _RPW)])

        def transform_and_fire(c, raw, sh, rows, gs):
            rsplat = jnp.full((16,), c, jnp.int32)
            for cc in range(_NCC):
                if cc < 8:
                    itv = plsc.load_gather(idx_a, [rsplat, lanes + cc * 16])
                else:
                    itv = plsc.load_gather(idx_b,
                                           [rsplat, lanes + (cc - 8) * 16])
                raw[pl.ds(cc * 16, 16)] = itv
                sh[pl.ds(cc * 16, 16)] = itv & 0xFFFFF
            pltpu.async_copy(table_hbm.at[sh.at[pl.ds(0, _GA)]],
                             rows.at[pl.ds(0, _GA)], gs)
            pltpu.async_copy(table_hbm.at[sh.at[pl.ds(_GA, _GB)]],
                             rows.at[pl.ds(_GA, _GB)], gs)

        def wait_gathers(sh, rows, gs):
            pltpu.make_async_copy(table_hbm.at[sh.at[pl.ds(0, _GA)]],
                                  rows.at[pl.ds(0, _GA)], gs).wait()
            pltpu.make_async_copy(table_hbm.at[sh.at[pl.ds(_GA, _GB)]],
                                  rows.at[pl.ds(_GA, _GB)], gs).wait()

        def compute(c, raw, rows, out_v):
            rsplat = jnp.full((16,), c, jnp.int32)
            u_lo = plsc.load_gather(u_v, [rsplat, lanes])
            u_hi = plsc.load_gather(u_v, [rsplat, lanes + 16])

            @plsc.parallel_loop(0, _NCC, 1)
            def cch(cc):
                base = cc * 16
                ridx = base + lanes
                itv = raw[pl.ds(base, 16)]
                sub = lax.shift_right_logical(itv, 15) & 96
                accs = [jnp.zeros((16,), jnp.float32) for _ in range(4)]
                for f in range(_F):
                    src = u_lo if f < 16 else u_hi
                    ub = src.at[jnp.full((16,), f % 16, jnp.int32)].get(
                        mode="promise_in_bounds")
                    vals = plsc.load_gather(rows, [ridx, sub + f])
                    accs[f % 4] = accs[f % 4] + ub * vals
                out_v[pl.ds(base, 16)] = ((accs[0] + accs[1])
                                          + (accs[2] + accs[3]))

        def slot(c, raw, sh, rows, out_v, raw_n, sh_n, rows_n,
                 gs_mine, gs_next, os_mine):
            pl.when(c + 1 < _NCH)(
                lambda: transform_and_fire(c + 1, raw_n, sh_n, rows_n,
                                           gs_next))
            wait_gathers(sh, rows, gs_mine)
            pl.when(c >= 2)(lambda: pltpu.make_async_copy(
                out_v.at[pl.ds(0, _L)],
                out_hbm.at[pl.ds(wid * _IPW, _L)], os_mine).wait())
            compute(c, raw, rows, out_v)
            pltpu.async_copy(out_v.at[pl.ds(0, _L)],
                             out_hbm.at[pl.ds(wid * _IPW + c * _L, _L)],
                             os_mine)

        transform_and_fire(0, raw0, sh0, rows0, gs0)

        def pair(i, carry):
            c = 2 * i
            slot(c, raw0, sh0, rows0, out0, raw1, sh1, rows1, gs0, gs1, os0)
            slot(c + 1, raw1, sh1, rows1, out1, raw0, sh0, rows0,
                 gs1, gs0, os1)
            return carry

        lax.fori_loop(0, _NCH // 2, pair, 0)
        pltpu.make_async_copy(out0.at[pl.ds(0, _L)],
                              out_hbm.at[pl.ds(wid * _IPW, _L)], os0).wait()
        pltpu.make_async_copy(out1.at[pl.ds(0, _L)],
                              out_hbm.at[pl.ds(wid * _IPW, _L)], os1).wait()

    return k(u_pad, items3d, table4)


def kernel(users, items, dow_emb, time_emb, sex_emb, age_emb, month_emb,
           day_emb, W, b, item_table):
    users = users.astype(jnp.int32)
    items = items.astype(jnp.int32)
    u_pad, items3d = _prep(users, items, dow_emb, time_emb, sex_emb, age_emb,
                           month_emb, day_emb, W, b)
    table4 = _sc_repack(item_table)
    out = _sc_dot(u_pad, items3d, table4)
    return out.reshape(_B, _L)
